# baseline (speedup 1.0000x reference)
import jax
import jax.numpy as jnp
from jax import lax
from jax.experimental import pallas as pl
from jax.experimental.pallas import tpu as pltpu
from jax.experimental.pallas import tpu_sc as plsc

POOL = 128
SEL = 5
PLEN = 5
DIM = 768
ITERS = 10
NQ = 1024
NPTS = NQ + 1
KPAD = 8
PD = PLEN * DIM
GROWS = NQ * SEL
GCHUNK = 256


def _mm_bt(a, b):
    return jax.lax.dot_general(a, b, (((1,), (1,)), ((), ())),
                               preferred_element_type=jnp.float32)


def _mm_at(a, b):
    return jax.lax.dot_general(a, b, (((0,), (0,)), ((), ())),
                               precision=jax.lax.Precision.HIGHEST,
                               preferred_element_type=jnp.float32)


def _main_kernel(xk_ref, x_ref, pb_ref, np_ref, dist_ref, idx_ref, pm_ref):
    x = x_ref[...]
    xk = xk_ref[...]
    k_rows = jax.lax.broadcasted_iota(jnp.int32, (KPAD, 1), 0)
    k_valid = (k_rows < 1).astype(jnp.float32)
    p2x = jnp.sum(x * x, axis=1, keepdims=True)
    p2k = jnp.sum(xk * xk, axis=1, keepdims=True)
    cols_x = jax.lax.broadcasted_iota(jnp.int32, (NQ, POOL), 1)
    cols_k = jax.lax.broadcasted_iota(jnp.int32, (KPAD, POOL), 1)
    ones_x = jnp.ones((NQ, 1), jnp.float32)
    ones_row_d = jnp.ones((1, DIM), jnp.float32)

    def assign_onehots(cents):
        c2_row = jax.lax.dot_general(
            ones_row_d, cents * cents, (((1,), (1,)), ((), ())),
            precision=jax.lax.Precision.HIGHEST,
            preferred_element_type=jnp.float32)
        dx = p2x - 2.0 * _mm_bt(x, cents) + c2_row
        mx = jnp.min(dx, axis=1, keepdims=True)
        ax = jnp.min(jnp.where(dx == mx, cols_x, POOL), axis=1, keepdims=True)
        oh_x = (cols_x == ax).astype(jnp.float32)
        dk = p2k - 2.0 * _mm_bt(xk, cents) + c2_row
        mk = jnp.min(dk, axis=1, keepdims=True)
        ak = jnp.min(jnp.where(dk == mk, cols_k, POOL), axis=1, keepdims=True)
        oh_k = (cols_k == ak).astype(jnp.float32) * k_valid
        return oh_x, oh_k

    def counts_of(oh_x, oh_k):
        c = _mm_at(oh_x, ones_x) + _mm_at(oh_k, k_valid)
        return c

    def body(_, cents):
        oh_x, oh_k = assign_onehots(cents)
        counts = counts_of(oh_x, oh_k)
        sums = _mm_at(oh_k, xk) + _mm_at(oh_x, x)
        return jnp.where(counts > 0.0,
                         sums / jnp.maximum(counts, 1.0), cents)

    cents0 = jnp.concatenate([xk[:1, :], x[:POOL - 1, :]], axis=0)
    cents = jax.lax.fori_loop(0, ITERS, body, cents0)
    oh_x, oh_k = assign_onehots(cents)
    denom = jnp.maximum(counts_of(oh_x, oh_k), 1.0)
    key_m = (_mm_at(oh_k, xk) + _mm_at(oh_x, x)) / denom
    for t in range(PLEN):
        pm_ref[:, t * DIM:(t + 1) * DIM] = (
            _mm_at(oh_k, pb_ref[:, t, :]) +
            _mm_at(oh_x, np_ref[:, t, :])) / denom

    xn = x / jnp.maximum(jnp.sqrt(p2x), 1e-8)
    kn = key_m / jnp.maximum(
        jnp.sqrt(jnp.sum(key_m * key_m, axis=1, keepdims=True)), 1e-8)
    dist = 1.0 - _mm_bt(xn, kn)
    vals, idxs = [], []
    for _ in range(SEL):
        m = jnp.min(dist, axis=1, keepdims=True)
        a_col = jnp.min(jnp.where(dist == m, cols_x, POOL),
                        axis=1, keepdims=True)
        vals.append(m)
        idxs.append(a_col)
        dist = jnp.where(cols_x == a_col, jnp.float32(jnp.inf), dist)
    dist_ref[...] = jnp.concatenate(vals, axis=1)
    idx_ref[...] = jnp.concatenate(idxs, axis=1)


NW = 32
BPW = GROWS // NW
CH = 8
NCH = BPW // CH


def _sc_gather_body(idx_hbm, pm_hbm, out_hbm,
                    idx0, idx1, buf0, buf1, sem0, sem1):
    wid = lax.axis_index("s") * 2 + lax.axis_index("c")
    base = wid * BPW
    idxs, bufs, sems = [idx0, idx1], [buf0, buf1], [sem0, sem1]
    copies = [None, None]
    for c in range(NCH):
        b = c & 1
        pltpu.sync_copy(idx_hbm.at[pl.ds(base + c * CH, CH)], idxs[b])
        copies[b] = pltpu.async_copy(pm_hbm.at[idxs[b]], bufs[b], sems[b])
        if c >= 1:
            pb = (c - 1) & 1
            copies[pb].wait()
            pltpu.sync_copy(bufs[pb],
                            out_hbm.at[pl.ds(base + (c - 1) * CH, CH)])
    lastb = (NCH - 1) & 1
    copies[lastb].wait()
    pltpu.sync_copy(bufs[lastb],
                    out_hbm.at[pl.ds(base + (NCH - 1) * CH, CH)])


_sc_gather = pl.kernel(
    _sc_gather_body,
    mesh=plsc.VectorSubcoreMesh(core_axis_name="c", subcore_axis_name="s"),
    out_type=jax.ShapeDtypeStruct((GROWS, PD), jnp.float32),
    scratch_types=[
        pltpu.VMEM((CH,), jnp.int32),
        pltpu.VMEM((CH,), jnp.int32),
        pltpu.VMEM((CH, PD), jnp.float32),
        pltpu.VMEM((CH, PD), jnp.float32),
        pltpu.SemaphoreType.DMA,
        pltpu.SemaphoreType.DMA,
    ],
)


def kernel(x, key_buf, prompts_buf, num_selections, new_prompts):
    xk_pad = jnp.pad(key_buf, ((0, KPAD - 1), (0, 0)))
    pb_pad = jnp.pad(prompts_buf, ((0, KPAD - 1), (0, 0), (0, 0)))

    dist_sel, topk, pm = pl.pallas_call(
        _main_kernel,
        out_shape=[
            jax.ShapeDtypeStruct((NQ, SEL), jnp.float32),
            jax.ShapeDtypeStruct((NQ, SEL), jnp.int32),
            jax.ShapeDtypeStruct((POOL, PD), jnp.float32),
        ],
    )(xk_pad, x, pb_pad, new_prompts)

    gathered = _sc_gather(topk.reshape(GROWS), pm)
    prompt = gathered.reshape(NQ, SEL, PLEN, DIM)
    return dist_sel, prompt


# speedup vs baseline: 1.0200x; 1.0200x over previous
import jax
import jax.numpy as jnp
from jax import lax
from jax.experimental import pallas as pl
from jax.experimental.pallas import tpu as pltpu
from jax.experimental.pallas import tpu_sc as plsc

POOL = 128
SEL = 5
PLEN = 5
DIM = 768
ITERS = 10
NQ = 1024
NPTS = NQ + 1
KPAD = 8
PD = PLEN * DIM
GROWS = NQ * SEL
GCHUNK = 256


def _mm_bt(a, b):
    return jax.lax.dot_general(a, b, (((1,), (1,)), ((), ())),
                               preferred_element_type=jnp.float32)


def _mm_at(a, b):
    return jax.lax.dot_general(a, b, (((0,), (0,)), ((), ())),
                               precision=jax.lax.Precision.HIGHEST,
                               preferred_element_type=jnp.float32)


def _mm_at_fast(a, b):
    return jax.lax.dot_general(a, b, (((0,), (0,)), ((), ())),
                               preferred_element_type=jnp.float32)


def _main_kernel(xk_ref, x_ref, pb_ref, np_ref, dist_ref, idx_ref, pm_ref):
    x = x_ref[...]
    xk = xk_ref[...]
    k_rows = jax.lax.broadcasted_iota(jnp.int32, (KPAD, 1), 0)
    k_valid = (k_rows < 1).astype(jnp.float32)
    p2x = jnp.sum(x * x, axis=1, keepdims=True)
    p2k = jnp.sum(xk * xk, axis=1, keepdims=True)
    cols_x = jax.lax.broadcasted_iota(jnp.int32, (NQ, POOL), 1)
    cols_k = jax.lax.broadcasted_iota(jnp.int32, (KPAD, POOL), 1)
    ones_x = jnp.ones((NQ, 1), jnp.float32)
    ones_row_d = jnp.ones((1, DIM), jnp.float32)

    def assign_onehots(cents):
        c2_row = jax.lax.dot_general(
            ones_row_d, cents * cents, (((1,), (1,)), ((), ())),
            precision=jax.lax.Precision.HIGHEST,
            preferred_element_type=jnp.float32)
        dx = p2x - 2.0 * _mm_bt(x, cents) + c2_row
        mx = jnp.min(dx, axis=1, keepdims=True)
        ax = jnp.min(jnp.where(dx == mx, cols_x, POOL), axis=1, keepdims=True)
        oh_x = (cols_x == ax).astype(jnp.float32)
        dk = p2k - 2.0 * _mm_bt(xk, cents) + c2_row
        mk = jnp.min(dk, axis=1, keepdims=True)
        ak = jnp.min(jnp.where(dk == mk, cols_k, POOL), axis=1, keepdims=True)
        oh_k = (cols_k == ak).astype(jnp.float32) * k_valid
        return oh_x, oh_k

    def counts_of(oh_x, oh_k):
        c = _mm_at(oh_x, ones_x) + _mm_at(oh_k, k_valid)
        return c

    def body(_, cents):
        oh_x, oh_k = assign_onehots(cents)
        counts = counts_of(oh_x, oh_k)
        sums = _mm_at(oh_k, xk) + _mm_at(oh_x, x)
        return jnp.where(counts > 0.0,
                         sums / jnp.maximum(counts, 1.0), cents)

    cents0 = jnp.concatenate([xk[:1, :], x[:POOL - 1, :]], axis=0)
    cents = jax.lax.fori_loop(0, ITERS, body, cents0)
    oh_x, oh_k = assign_onehots(cents)
    denom = jnp.maximum(counts_of(oh_x, oh_k), 1.0)
    key_m = (_mm_at(oh_k, xk) + _mm_at(oh_x, x)) / denom
    for t in range(PLEN):
        pm_ref[:, t * DIM:(t + 1) * DIM] = (
            _mm_at_fast(oh_k, pb_ref[:, t, :]) +
            _mm_at_fast(oh_x, np_ref[:, t, :])) / denom

    xn = x / jnp.maximum(jnp.sqrt(p2x), 1e-8)
    kn = key_m / jnp.maximum(
        jnp.sqrt(jnp.sum(key_m * key_m, axis=1, keepdims=True)), 1e-8)
    dist = 1.0 - _mm_bt(xn, kn)
    vals, idxs = [], []
    for _ in range(SEL):
        m = jnp.min(dist, axis=1, keepdims=True)
        a_col = jnp.min(jnp.where(dist == m, cols_x, POOL),
                        axis=1, keepdims=True)
        vals.append(m)
        idxs.append(a_col)
        dist = jnp.where(cols_x == a_col, jnp.float32(jnp.inf), dist)
    dist_ref[...] = jnp.concatenate(vals, axis=1)
    idx_ref[...] = jnp.concatenate(idxs, axis=1)


NW = 32
BPW = GROWS // NW
CH = 16
NCH = BPW // CH


def _sc_gather_body(idx_hbm, pm_hbm, out_hbm,
                    idx_all, buf0, buf1, sem0, sem1):
    wid = lax.axis_index("s") * 2 + lax.axis_index("c")
    base = wid * BPW
    pltpu.sync_copy(idx_hbm.at[pl.ds(base, BPW)], idx_all)
    bufs, sems = [buf0, buf1], [sem0, sem1]
    copies = [None, None]
    for c in range(NCH):
        b = c & 1
        copies[b] = pltpu.async_copy(
            pm_hbm.at[idx_all.at[pl.ds(c * CH, CH)]], bufs[b], sems[b])
        if c >= 1:
            pb = (c - 1) & 1
            copies[pb].wait()
            pltpu.sync_copy(bufs[pb],
                            out_hbm.at[pl.ds(base + (c - 1) * CH, CH)])
    lastb = (NCH - 1) & 1
    copies[lastb].wait()
    pltpu.sync_copy(bufs[lastb],
                    out_hbm.at[pl.ds(base + (NCH - 1) * CH, CH)])


_sc_gather = pl.kernel(
    _sc_gather_body,
    mesh=plsc.VectorSubcoreMesh(core_axis_name="c", subcore_axis_name="s"),
    out_type=jax.ShapeDtypeStruct((GROWS, PD), jnp.float32),
    scratch_types=[
        pltpu.VMEM((BPW,), jnp.int32),
        pltpu.VMEM((CH, PD), jnp.float32),
        pltpu.VMEM((CH, PD), jnp.float32),
        pltpu.SemaphoreType.DMA,
        pltpu.SemaphoreType.DMA,
    ],
)


def kernel(x, key_buf, prompts_buf, num_selections, new_prompts):
    xk_pad = jnp.pad(key_buf, ((0, KPAD - 1), (0, 0)))
    pb_pad = jnp.pad(prompts_buf, ((0, KPAD - 1), (0, 0), (0, 0)))

    dist_sel, topk, pm = pl.pallas_call(
        _main_kernel,
        out_shape=[
            jax.ShapeDtypeStruct((NQ, SEL), jnp.float32),
            jax.ShapeDtypeStruct((NQ, SEL), jnp.int32),
            jax.ShapeDtypeStruct((POOL, PD), jnp.float32),
        ],
    )(xk_pad, x, pb_pad, new_prompts)

    gathered = _sc_gather(topk.reshape(GROWS), pm)
    prompt = gathered.reshape(NQ, SEL, PLEN, DIM)
    return dist_sel, prompt


# speedup vs baseline: 1.0237x; 1.0037x over previous
import jax
import jax.numpy as jnp
from jax import lax
from jax.experimental import pallas as pl
from jax.experimental.pallas import tpu as pltpu
from jax.experimental.pallas import tpu_sc as plsc

POOL = 128
SEL = 5
PLEN = 5
DIM = 768
ITERS = 10
NQ = 1024
KPAD = 8
PD = PLEN * DIM
GROWS = NQ * SEL


def _mm_bt(a, b):
    return jax.lax.dot_general(a, b, (((1,), (1,)), ((), ())),
                               preferred_element_type=jnp.float32)


def _mm_at(a, b):
    return jax.lax.dot_general(a, b, (((0,), (0,)), ((), ())),
                               precision=jax.lax.Precision.HIGHEST,
                               preferred_element_type=jnp.float32)


def _mm_at_fast(a, b):
    return jax.lax.dot_general(a, b, (((0,), (0,)), ((), ())),
                               preferred_element_type=jnp.float32)


def _main_kernel(xk_ref, x_ref, pb_ref, np_ref, dist_ref, idx_ref, pm_ref):
    x = x_ref[...]
    xk = xk_ref[...]
    k_rows = jax.lax.broadcasted_iota(jnp.int32, (KPAD, 1), 0)
    k_valid = (k_rows < 1).astype(jnp.float32)
    p2x = jnp.sum(x * x, axis=1, keepdims=True)
    p2k = jnp.sum(xk * xk, axis=1, keepdims=True)
    cols_x = jax.lax.broadcasted_iota(jnp.int32, (NQ, POOL), 1)
    cols_k = jax.lax.broadcasted_iota(jnp.int32, (KPAD, POOL), 1)
    ones_x = jnp.ones((NQ, 1), jnp.float32)
    ones_row_d = jnp.ones((1, DIM), jnp.float32)

    def assign_onehots(cents):
        c2_row = jax.lax.dot_general(
            ones_row_d, cents * cents, (((1,), (1,)), ((), ())),
            precision=jax.lax.Precision.HIGHEST,
            preferred_element_type=jnp.float32)
        dx = p2x - 2.0 * _mm_bt(x, cents) + c2_row
        mx = jnp.min(dx, axis=1, keepdims=True)
        ax = jnp.min(jnp.where(dx == mx, cols_x, POOL), axis=1, keepdims=True)
        oh_x = (cols_x == ax).astype(jnp.float32)
        dk = p2k - 2.0 * _mm_bt(xk, cents) + c2_row
        mk = jnp.min(dk, axis=1, keepdims=True)
        ak = jnp.min(jnp.where(dk == mk, cols_k, POOL), axis=1, keepdims=True)
        oh_k = (cols_k == ak).astype(jnp.float32) * k_valid
        return oh_x, oh_k

    def counts_of(oh_x, oh_k):
        c = _mm_at(oh_x, ones_x) + _mm_at(oh_k, k_valid)
        return c

    def body(_, cents):
        oh_x, oh_k = assign_onehots(cents)
        counts = counts_of(oh_x, oh_k)
        sums = _mm_at(oh_k, xk) + _mm_at(oh_x, x)
        return jnp.where(counts > 0.0,
                         sums / jnp.maximum(counts, 1.0), cents)

    cents0 = jnp.concatenate([xk[:1, :], x[:POOL - 1, :]], axis=0)
    cents = jax.lax.fori_loop(0, ITERS, body, cents0)
    oh_x, oh_k = assign_onehots(cents)
    denom = jnp.maximum(counts_of(oh_x, oh_k), 1.0)
    key_m = (_mm_at(oh_k, xk) + _mm_at(oh_x, x)) / denom
    for t in range(PLEN):
        pm_ref[:, t * DIM:(t + 1) * DIM] = (
            _mm_at_fast(oh_k, pb_ref[:, t, :]) +
            _mm_at_fast(oh_x, np_ref[:, t, :])) / denom

    xn = x / jnp.maximum(jnp.sqrt(p2x), 1e-8)
    kn = key_m / jnp.maximum(
        jnp.sqrt(jnp.sum(key_m * key_m, axis=1, keepdims=True)), 1e-8)
    dist = 1.0 - _mm_bt(xn, kn)
    vals, idxs = [], []
    for _ in range(SEL):
        m = jnp.min(dist, axis=1, keepdims=True)
        a_col = jnp.min(jnp.where(dist == m, cols_x, POOL),
                        axis=1, keepdims=True)
        vals.append(m)
        idxs.append(a_col)
        dist = jnp.where(cols_x == a_col, jnp.float32(jnp.inf), dist)
    dist_ref[...] = jnp.concatenate(vals, axis=1)
    idx_ref[...] = jnp.concatenate(idxs, axis=1)


NW = 32
BPW = GROWS // NW
CH = 16
NCH = BPW // CH


def _sc_gather_body(idx_hbm, pm_hbm, out_hbm,
                    idx_all, buf0, buf1, sem0, sem1):
    wid = lax.axis_index("s") * 2 + lax.axis_index("c")
    base = wid * BPW
    pltpu.sync_copy(idx_hbm.at[pl.ds(base, BPW)], idx_all)
    bufs, sems = [buf0, buf1], [sem0, sem1]
    copies = [None, None]
    for c in range(NCH):
        b = c & 1
        copies[b] = pltpu.async_copy(
            pm_hbm.at[idx_all.at[pl.ds(c * CH, CH)]], bufs[b], sems[b])
        if c >= 1:
            pb = (c - 1) & 1
            copies[pb].wait()
            pltpu.sync_copy(bufs[pb],
                            out_hbm.at[pl.ds(base + (c - 1) * CH, CH)])
    lastb = (NCH - 1) & 1
    copies[lastb].wait()
    pltpu.sync_copy(bufs[lastb],
                    out_hbm.at[pl.ds(base + (NCH - 1) * CH, CH)])


_sc_gather = pl.kernel(
    _sc_gather_body,
    mesh=plsc.VectorSubcoreMesh(core_axis_name="c", subcore_axis_name="s"),
    out_type=jax.ShapeDtypeStruct((GROWS, PD), jnp.float32),
    scratch_types=[
        pltpu.VMEM((BPW,), jnp.int32),
        pltpu.VMEM((CH, PD), jnp.float32),
        pltpu.VMEM((CH, PD), jnp.float32),
        pltpu.SemaphoreType.DMA,
        pltpu.SemaphoreType.DMA,
    ],
)


def kernel(x, key_buf, prompts_buf, num_selections, new_prompts):
    xk_pad = jnp.pad(key_buf, ((0, KPAD - 1), (0, 0)))
    pb_pad = jnp.pad(prompts_buf, ((0, KPAD - 1), (0, 0), (0, 0)))

    dist_sel, topk, pm = pl.pallas_call(
        _main_kernel,
        out_shape=[
            jax.ShapeDtypeStruct((NQ, SEL), jnp.float32),
            jax.ShapeDtypeStruct((NQ, SEL), jnp.int32),
            jax.ShapeDtypeStruct((POOL, PD), jnp.float32),
        ],
    )(xk_pad, x, pb_pad, new_prompts)

    gathered = _sc_gather(topk.reshape(GROWS), pm)
    prompt = gathered.reshape(NQ, SEL, PLEN, DIM)
    return dist_sel, prompt
